# Initial kernel scaffold; baseline (speedup 1.0000x reference)
import jax
import jax.numpy as jnp
from jax.experimental import pallas as pl


def kernel(mrope_position_ids_padding, mrope_position_deltas, inv_freq):
    raise NotImplementedError("write your pallas kernel here")


# speedup vs baseline: 35.9394x; 35.9394x over previous
import functools
import math

import jax
import jax.numpy as jnp
from jax import lax
from jax.experimental import pallas as pl
from jax.experimental.pallas import tpu as pltpu
from jax.experimental.pallas import tpu_sc as plsc

MAX_POS = 8192
BATCH = 4
WIDTHS = (32, 48, 48)
COL_OFF = (0, 32, 80)

QN = 128
MN = 64

NUM_WORKERS = 32
POS_TOTAL = BATCH * MAX_POS
P_PER_W = POS_TOTAL // NUM_WORKERS
CHUNK = 128
N_CHUNKS = P_PER_W // CHUNK


def _table_body(invf2_ref, t0_ref, t1_ref, t2_ref):
    invf2 = invf2_ref[...]
    qa = lax.broadcasted_iota(jnp.int32, (QN, 128), 0).astype(jnp.float32) * 64.0
    ang_a = qa * invf2
    ca, sa = jnp.cos(ang_a), jnp.sin(ang_a)
    mb = lax.broadcasted_iota(jnp.int32, (MN, 128), 0).astype(jnp.float32)
    k = (lax.broadcasted_iota(jnp.int32, (MN, 128), 1) % 2).astype(jnp.float32)
    ang_b = mb * invf2 - k * (math.pi / 2.0)
    cb, sb = jnp.cos(ang_b), jnp.sin(ang_b)
    full = ca[:, None, :] * cb[None, :, :] - sa[:, None, :] * sb[None, :, :]
    full = full.reshape(QN * MN, 128)
    t0_ref[...] = full[:, COL_OFF[0]:COL_OFF[0] + WIDTHS[0]]
    t1_ref[...] = full[:, COL_OFF[1]:COL_OFF[1] + WIDTHS[1]]
    t2_ref[...] = full[:, COL_OFF[2]:COL_OFF[2] + WIDTHS[2]]


def _build_tables(invf2):
    return pl.pallas_call(
        _table_body,
        out_shape=tuple(
            jax.ShapeDtypeStruct((MAX_POS, w), jnp.float32) for w in WIDTHS
        ),
    )(invf2)


@functools.partial(
    pl.kernel,
    mesh=plsc.VectorSubcoreMesh(core_axis_name="c", subcore_axis_name="s"),
    out_type=jax.ShapeDtypeStruct((POS_TOTAL, 128), jnp.float32),
    compiler_params=pltpu.CompilerParams(use_tc_tiling_on_sc=False),
    scratch_types=[
        pltpu.VMEM((3 * P_PER_W,), jnp.int32),
        pltpu.VMEM((CHUNK, WIDTHS[0]), jnp.float32),
        pltpu.VMEM((CHUNK, WIDTHS[1]), jnp.float32),
        pltpu.VMEM((CHUNK, WIDTHS[2]), jnp.float32),
        pltpu.SemaphoreType.DMA,
    ],
)
def _sc_gather(ids_hbm, t0_hbm, t1_hbm, t2_hbm, out_hbm, idx_v, g0, g1, g2, sem):
    c = lax.axis_index("c")
    s = lax.axis_index("s")
    wid = s * 2 + c
    b = wid // (MAX_POS // P_PER_W)
    base = wid * P_PER_W
    p0 = base - b * MAX_POS
    for sct in range(3):
        pltpu.sync_copy(
            ids_hbm.at[pl.ds((b * 3 + sct) * MAX_POS + p0, P_PER_W)],
            idx_v.at[pl.ds(sct * P_PER_W, P_PER_W)])
    for i in range(N_CHUNKS):
        cps = []
        for sct, (t_hbm, g) in enumerate(((t0_hbm, g0), (t1_hbm, g1), (t2_hbm, g2))):
            idx = idx_v.at[pl.ds(sct * P_PER_W + i * CHUNK, CHUNK)]
            cps.append(pltpu.async_copy(t_hbm.at[idx], g, sem))
        for cp in cps:
            cp.wait()
        row = base + i * CHUNK
        pltpu.sync_copy(g0, out_hbm.at[pl.ds(row, CHUNK), pl.ds(COL_OFF[0], WIDTHS[0])])
        pltpu.sync_copy(g1, out_hbm.at[pl.ds(row, CHUNK), pl.ds(COL_OFF[1], WIDTHS[1])])
        pltpu.sync_copy(g2, out_hbm.at[pl.ds(row, CHUNK), pl.ds(COL_OFF[2], WIDTHS[2])])


def kernel(mrope_position_ids_padding, mrope_position_deltas, inv_freq):
    invf2 = jnp.repeat(inv_freq, 2).reshape(1, 128)
    t0, t1, t2 = _build_tables(invf2)
    ids_flat = mrope_position_ids_padding.reshape(BATCH * 3 * MAX_POS)
    out = _sc_gather(ids_flat, t0, t1, t2)
    cc = out.reshape(BATCH, MAX_POS * 128)
    return (cc, mrope_position_deltas)


# speedup vs baseline: 38.6709x; 1.0760x over previous
import functools
import math

import jax
import jax.numpy as jnp
from jax import lax
from jax.experimental import pallas as pl
from jax.experimental.pallas import tpu as pltpu
from jax.experimental.pallas import tpu_sc as plsc

MAX_POS = 8192
BATCH = 4
WIDTHS = (32, 48, 48)
COL_OFF = (0, 32, 80)

QN = 128
MN = 64

NUM_WORKERS = 32
POS_TOTAL = BATCH * MAX_POS
P_PER_W = POS_TOTAL // NUM_WORKERS
CHUNK = 128
N_CHUNKS = P_PER_W // CHUNK
NBUF = 3


def _table_body(invf2_ref, t0_ref, t1_ref, t2_ref):
    invf2 = invf2_ref[...]
    qa = lax.broadcasted_iota(jnp.int32, (QN, 128), 0).astype(jnp.float32) * 64.0
    ang_a = qa * invf2
    ca, sa = jnp.cos(ang_a), jnp.sin(ang_a)
    mb = lax.broadcasted_iota(jnp.int32, (MN, 128), 0).astype(jnp.float32)
    k = (lax.broadcasted_iota(jnp.int32, (MN, 128), 1) % 2).astype(jnp.float32)
    ang_b = mb * invf2 - k * (math.pi / 2.0)
    cb, sb = jnp.cos(ang_b), jnp.sin(ang_b)
    full = ca[:, None, :] * cb[None, :, :] - sa[:, None, :] * sb[None, :, :]
    full = full.reshape(QN * MN, 128)
    t0_ref[...] = full[:, COL_OFF[0]:COL_OFF[0] + WIDTHS[0]]
    t1_ref[...] = full[:, COL_OFF[1]:COL_OFF[1] + WIDTHS[1]]
    t2_ref[...] = full[:, COL_OFF[2]:COL_OFF[2] + WIDTHS[2]]


def _build_tables(invf2):
    return pl.pallas_call(
        _table_body,
        out_shape=tuple(
            jax.ShapeDtypeStruct((MAX_POS, w), jnp.float32) for w in WIDTHS
        ),
    )(invf2)


@functools.partial(
    pl.kernel,
    mesh=plsc.VectorSubcoreMesh(core_axis_name="c", subcore_axis_name="s"),
    out_type=jax.ShapeDtypeStruct((POS_TOTAL, 128), jnp.float32),
    compiler_params=pltpu.CompilerParams(use_tc_tiling_on_sc=False),
    scratch_types=[
        pltpu.VMEM((3 * P_PER_W,), jnp.int32),
        pltpu.VMEM((NBUF, CHUNK, WIDTHS[0]), jnp.float32),
        pltpu.VMEM((NBUF, CHUNK, WIDTHS[1]), jnp.float32),
        pltpu.VMEM((NBUF, CHUNK, WIDTHS[2]), jnp.float32),
        pltpu.SemaphoreType.DMA,
        pltpu.SemaphoreType.DMA,
    ],
)
def _sc_gather(ids_hbm, t0_hbm, t1_hbm, t2_hbm, out_hbm, idx_v, g0, g1, g2,
               gsem, ssem):
    c = lax.axis_index("c")
    s = lax.axis_index("s")
    wid = s * 2 + c
    b = wid // (MAX_POS // P_PER_W)
    base = wid * P_PER_W
    p0 = base - b * MAX_POS
    for sct in range(3):
        pltpu.sync_copy(
            ids_hbm.at[pl.ds((b * 3 + sct) * MAX_POS + p0, P_PER_W)],
            idx_v.at[pl.ds(sct * P_PER_W, P_PER_W)])

    tables = (t0_hbm, t1_hbm, t2_hbm)
    bufs = (g0, g1, g2)

    def start_gathers(i):
        slot = i % NBUF
        cps = []
        for sct in range(3):
            idx = idx_v.at[pl.ds(sct * P_PER_W + i * CHUNK, CHUNK)]
            cps.append(
                pltpu.async_copy(tables[sct].at[idx], bufs[sct].at[slot], gsem))
        return cps

    def start_scatters(i):
        slot = i % NBUF
        row = base + i * CHUNK
        cps = []
        for sct in range(3):
            cps.append(pltpu.async_copy(
                bufs[sct].at[slot],
                out_hbm.at[pl.ds(row, CHUNK), pl.ds(COL_OFF[sct], WIDTHS[sct])],
                ssem))
        return cps

    scat = [None] * NBUF
    g_infl = start_gathers(0)
    for i in range(N_CHUNKS):
        if i + 1 < N_CHUNKS:
            slot = (i + 1) % NBUF
            if scat[slot] is not None:
                for cp in scat[slot]:
                    cp.wait()
                scat[slot] = None
            nxt = start_gathers(i + 1)
        else:
            nxt = None
        for cp in g_infl:
            cp.wait()
        scat[i % NBUF] = start_scatters(i)
        g_infl = nxt
    for cps in scat:
        if cps is not None:
            for cp in cps:
                cp.wait()


def kernel(mrope_position_ids_padding, mrope_position_deltas, inv_freq):
    invf2 = jnp.repeat(inv_freq, 2).reshape(1, 128)
    t0, t1, t2 = _build_tables(invf2)
    ids_flat = mrope_position_ids_padding.reshape(BATCH * 3 * MAX_POS)
    out = _sc_gather(ids_flat, t0, t1, t2)
    cc = out.reshape(BATCH, MAX_POS * 128)
    return (cc, mrope_position_deltas)


# speedup vs baseline: 39.3233x; 1.0169x over previous
import functools
import math

import jax
import jax.numpy as jnp
from jax import lax
from jax.experimental import pallas as pl
from jax.experimental.pallas import tpu as pltpu
from jax.experimental.pallas import tpu_sc as plsc

MAX_POS = 8192
BATCH = 4
WIDTHS = (32, 48, 48)
COL_OFF = (0, 32, 80)

QN = 128
MN = 64

NUM_WORKERS = 32
POS_TOTAL = BATCH * MAX_POS
P_PER_W = POS_TOTAL // NUM_WORKERS
CHUNK = 256
N_CHUNKS = P_PER_W // CHUNK
NBUF = 3


def _table_body(invf2_ref, t0_ref, t1_ref, t2_ref):
    invf2 = invf2_ref[...]
    qa = lax.broadcasted_iota(jnp.int32, (QN, 128), 0).astype(jnp.float32) * 64.0
    ang_a = qa * invf2
    ca, sa = jnp.cos(ang_a), jnp.sin(ang_a)
    mb = lax.broadcasted_iota(jnp.int32, (MN, 128), 0).astype(jnp.float32)
    k = (lax.broadcasted_iota(jnp.int32, (MN, 128), 1) % 2).astype(jnp.float32)
    ang_b = mb * invf2 - k * (math.pi / 2.0)
    cb, sb = jnp.cos(ang_b), jnp.sin(ang_b)
    full = ca[:, None, :] * cb[None, :, :] - sa[:, None, :] * sb[None, :, :]
    full = full.reshape(QN * MN, 128)
    t0_ref[...] = full[:, COL_OFF[0]:COL_OFF[0] + WIDTHS[0]]
    t1_ref[...] = full[:, COL_OFF[1]:COL_OFF[1] + WIDTHS[1]]
    t2_ref[...] = full[:, COL_OFF[2]:COL_OFF[2] + WIDTHS[2]]


def _build_tables(invf2):
    return pl.pallas_call(
        _table_body,
        out_shape=tuple(
            jax.ShapeDtypeStruct((MAX_POS, w), jnp.float32) for w in WIDTHS
        ),
    )(invf2)


@functools.partial(
    pl.kernel,
    mesh=plsc.VectorSubcoreMesh(core_axis_name="c", subcore_axis_name="s"),
    out_type=jax.ShapeDtypeStruct((POS_TOTAL, 128), jnp.float32),
    compiler_params=pltpu.CompilerParams(use_tc_tiling_on_sc=False),
    scratch_types=[
        pltpu.VMEM((3 * P_PER_W,), jnp.int32),
        pltpu.VMEM((NBUF, CHUNK, WIDTHS[0]), jnp.float32),
        pltpu.VMEM((NBUF, CHUNK, WIDTHS[1]), jnp.float32),
        pltpu.VMEM((NBUF, CHUNK, WIDTHS[2]), jnp.float32),
        pltpu.SemaphoreType.DMA,
        pltpu.SemaphoreType.DMA,
        pltpu.SemaphoreType.DMA,
        pltpu.SemaphoreType.DMA,
        pltpu.SemaphoreType.DMA,
        pltpu.SemaphoreType.DMA,
        pltpu.SemaphoreType.DMA,
    ],
)
def _sc_gather(ids_hbm, t0_hbm, t1_hbm, t2_hbm, out_hbm, idx_v, g0, g1, g2,
               idsem, gs0, gs1, gs2, ss0, ss1, ss2):
    c = lax.axis_index("c")
    s = lax.axis_index("s")
    wid = s * 2 + c
    b = wid // (MAX_POS // P_PER_W)
    base = wid * P_PER_W
    p0 = base - b * MAX_POS
    id_cps = [
        pltpu.async_copy(
            ids_hbm.at[pl.ds((b * 3 + sct) * MAX_POS + p0, P_PER_W)],
            idx_v.at[pl.ds(sct * P_PER_W, P_PER_W)], idsem)
        for sct in range(3)
    ]
    for cp in id_cps:
        cp.wait()

    tables = (t0_hbm, t1_hbm, t2_hbm)
    bufs = (g0, g1, g2)
    gsems = (gs0, gs1, gs2)
    ssems = (ss0, ss1, ss2)

    def start_gathers(i):
        slot = i % NBUF
        cps = []
        for sct in range(3):
            idx = idx_v.at[pl.ds(sct * P_PER_W + i * CHUNK, CHUNK)]
            cps.append(pltpu.async_copy(
                tables[sct].at[idx], bufs[sct].at[slot], gsems[slot]))
        return cps

    def start_scatters(i):
        slot = i % NBUF
        row = base + i * CHUNK
        cps = []
        for sct in range(3):
            cps.append(pltpu.async_copy(
                bufs[sct].at[slot],
                out_hbm.at[pl.ds(row, CHUNK), pl.ds(COL_OFF[sct], WIDTHS[sct])],
                ssems[slot]))
        return cps

    scat = [None] * NBUF
    g_infl = start_gathers(0)
    for i in range(N_CHUNKS):
        if i + 1 < N_CHUNKS:
            slot = (i + 1) % NBUF
            if scat[slot] is not None:
                for cp in scat[slot]:
                    cp.wait()
                scat[slot] = None
            nxt = start_gathers(i + 1)
        else:
            nxt = None
        for cp in g_infl:
            cp.wait()
        scat[i % NBUF] = start_scatters(i)
        g_infl = nxt
    for cps in scat:
        if cps is not None:
            for cp in cps:
                cp.wait()


def kernel(mrope_position_ids_padding, mrope_position_deltas, inv_freq):
    invf2 = jnp.repeat(inv_freq, 2).reshape(1, 128)
    t0, t1, t2 = _build_tables(invf2)
    ids_flat = mrope_position_ids_padding.reshape(BATCH * 3 * MAX_POS)
    out = _sc_gather(ids_flat, t0, t1, t2)
    cc = out.reshape(BATCH, MAX_POS * 128)
    return (cc, mrope_position_deltas)
